# Initial kernel scaffold; baseline (speedup 1.0000x reference)
import jax
import jax.numpy as jnp
from jax.experimental import pallas as pl


def kernel(input_ids, table):
    raise NotImplementedError("write your pallas kernel here")


# speedup vs baseline: 1.7646x; 1.7646x over previous
import functools

import jax
import jax.numpy as jnp
from jax import lax
from jax.experimental import pallas as pl
from jax.experimental.pallas import tpu as pltpu
from jax.experimental.pallas import tpu_sc as plsc

NC = 2
NS = 16
NW = NC * NS

K = 8
NB = 2


@functools.partial(jax.jit, static_argnums=())
def _gather_rows(ids, table):
    B, = ids.shape
    V, D = table.shape
    b_per_w = B // NW
    nchunk = b_per_w // K

    mesh = plsc.VectorSubcoreMesh(core_axis_name="c", subcore_axis_name="s")

    @functools.partial(
        pl.kernel,
        out_type=jax.ShapeDtypeStruct((B, D), jnp.float32),
        mesh=mesh,
        scratch_types=[
            pltpu.VMEM((b_per_w,), jnp.int32),
            pltpu.VMEM((NB, K, D), jnp.float32),
            pltpu.SemaphoreType.DMA,
            pltpu.SemaphoreType.DMA,
            pltpu.SemaphoreType.DMA,
            pltpu.SemaphoreType.DMA,
        ],
    )
    def body(ids_hbm, table_hbm, out_hbm, idx_v, bufs, g0, g1, w0, w1):
        gsems = (g0, g1)
        wsems = (w0, w1)
        wid = lax.axis_index("s") * NC + lax.axis_index("c")
        base = wid * b_per_w

        pltpu.sync_copy(ids_hbm.at[pl.ds(base, b_per_w)], idx_v)

        for b in range(NB):
            pltpu.async_copy(
                table_hbm.at[idx_v.at[pl.ds(b * K, K)]], bufs.at[b], gsems[b]
            )

        @pl.loop(0, nchunk, step=NB)
        def _(c0):
            for b in range(NB):
                c = c0 + b
                pltpu.make_async_copy(
                    table_hbm.at[idx_v.at[pl.ds(c * K, K)]],
                    bufs.at[b],
                    gsems[b],
                ).wait()
                out_slice = out_hbm.at[pl.ds(base + c * K, K)]
                pltpu.async_copy(bufs.at[b], out_slice, wsems[b])
                pltpu.make_async_copy(bufs.at[b], out_slice, wsems[b]).wait()

                @pl.when(c + NB < nchunk)
                def _():
                    pltpu.async_copy(
                        table_hbm.at[idx_v.at[pl.ds((c + NB) * K, K)]],
                        bufs.at[b],
                        gsems[b],
                    )

    return body(ids, table)


def kernel(input_ids, table):
    ids = input_ids.reshape(-1).astype(jnp.int32)
    out = _gather_rows(ids, table)
    return out.reshape(input_ids.shape + (table.shape[1],))
